# baseline (speedup 1.0000x reference)
import functools

import jax
import jax.numpy as jnp
from jax import lax
from jax.experimental import pallas as pl
from jax.experimental.pallas import tpu as pltpu
from jax.experimental.pallas import tpu_sc as plsc

F32 = jnp.float32
_CH = 80


def _dot(a, b):
    return jnp.dot(a, b, preferred_element_type=F32,
                   precision=jax.lax.Precision.DEFAULT)


def _r8(v):
    v = v.reshape(1, -1)
    return jnp.broadcast_to(v, (8, v.shape[1]))


def _node_pre(x, u8, Wa, Wb, Wna, Wd, b0, bn0):
    N, D = x.shape
    BN = 400
    nb = N // BN

    def kern(x_ref, u_ref, wa_ref, wb_ref, wna_ref, wd_ref, b0_ref, bn0_ref,
             r1_ref, rp_ref, c_ref):
        xb = x_ref[...]
        ce = _dot(u_ref[...], wd_ref[...]) + b0_ref[...]
        r1_ref[...] = _dot(xb, wa_ref[...]) + ce[0:1, :]
        rp_ref[...] = _dot(xb, wna_ref[...]) + bn0_ref[0:1, :]
        c_ref[...] = _dot(xb, wb_ref[...])

    full = lambda s: pl.BlockSpec(s, lambda i: (0, 0))
    blk = pl.BlockSpec((BN, D), lambda i: (i, 0))
    return pl.pallas_call(
        kern,
        grid=(nb,),
        in_specs=[
            blk,
            full(u8.shape), full(Wa.shape), full(Wb.shape), full(Wna.shape),
            full(Wd.shape), full(b0.shape), full(bn0.shape),
        ],
        out_specs=[blk, blk, blk],
        out_shape=[jax.ShapeDtypeStruct((N, D), F32)] * 3,
    )(x, u8, Wa, Wb, Wna, Wd, b0, bn0)


def _edge(g1, gp, gc, e, Wc, W1, b1, W2, b2, Wnb):
    E, H = g1.shape
    ED = e.shape[1]
    BE = 512
    nb = E // BE

    def kern(g1_ref, gp_ref, gc_ref, e_ref, wc_ref, w1_ref, b1_ref, w2_ref,
             b2_ref, wnb_ref, m1_ref, eo_ref, es_ref):
        h1 = jnp.maximum(
            g1_ref[...] + gc_ref[...] + _dot(e_ref[...], wc_ref[...]), 0.0)
        h2 = jnp.maximum(_dot(h1, w1_ref[...]) + b1_ref[0:1, :], 0.0)
        en = _dot(h2, w2_ref[...]) + b2_ref[0:1, :]
        m1_ref[...] = jnp.maximum(gp_ref[...] + _dot(en, wnb_ref[...]), 0.0)
        eo_ref[...] = en

        @pl.when(pl.program_id(0) == 0)
        def _():
            es_ref[...] = jnp.zeros_like(es_ref)

        es_ref[...] += jnp.broadcast_to(
            jnp.sum(en, axis=0, keepdims=True), (8, ED))

    full = lambda s: pl.BlockSpec(s, lambda i: (0, 0))
    blkh = pl.BlockSpec((BE, H), lambda i: (i, 0))
    return pl.pallas_call(
        kern,
        grid=(nb,),
        in_specs=[
            blkh, blkh, blkh,
            pl.BlockSpec((BE, ED), lambda i: (i, 0)),
            full(Wc.shape), full(W1.shape), full(b1.shape), full(W2.shape),
            full(b2.shape), full(Wnb.shape),
        ],
        out_specs=[
            blkh,
            pl.BlockSpec((BE, ED), lambda i: (i, 0)),
            full((8, ED)),
        ],
        out_shape=[
            jax.ShapeDtypeStruct((E, H), F32),
            jax.ShapeDtypeStruct((E, ED), F32),
            jax.ShapeDtypeStruct((8, ED), F32),
        ],
    )(g1, gp, gc, e, Wc, W1, b1, W2, b2, Wnb)


def _node_post(x, s0, s1, c0, c1, u8, Wn1, bn1, W2x, W2a, W2u, b20, W21, b21,
               Ws1, bs1, Ws2p, bs2p, last):
    N, D = x.shape
    BN = 400
    nb = N // BN

    def kern(x_ref, s0_ref, s1_ref, c0_ref, c1_ref, u_ref, wn1_ref, bn1_ref,
             w2x_ref, w2a_ref, w2u_ref, b20_ref, w21_ref, b21_ref, ws1_ref,
             bs1_ref, ws2_ref, bs2_ref, *outs):
        if last:
            xo_ref, xs_ref, sh_ref = outs
        else:
            xo_ref, xs_ref = outs
        xb = x_ref[...]
        cnt = c0_ref[...][:, 0:1] + c1_ref[...][:, 0:1]
        s = s0_ref[...] + s1_ref[...]
        denom = jnp.maximum(cnt, 1.0)
        agg = (_dot(s, wn1_ref[...]) + cnt * bn1_ref[0:1, :]) / denom
        cu = _dot(u_ref[...], w2u_ref[...]) + b20_ref[...]
        t = jnp.maximum(
            _dot(xb, w2x_ref[...]) + _dot(agg, w2a_ref[...]) + cu[0:1, :], 0.0)
        xn = _dot(t, w21_ref[...]) + b21_ref[0:1, :]
        xo_ref[...] = xn

        @pl.when(pl.program_id(0) == 0)
        def _():
            xs_ref[...] = jnp.zeros_like(xs_ref)

        xs_ref[...] += jnp.broadcast_to(
            jnp.sum(xn, axis=0, keepdims=True), (8, D))

        if last:
            t2 = jnp.maximum(_dot(xn, ws1_ref[...]) + bs1_ref[0:1, :], 0.0)
            sh_ref[...] = _dot(t2, ws2_ref[...]) + bs2_ref[0:1, :]

    full = lambda s: pl.BlockSpec(s, lambda i: (0, 0))
    blk = lambda w: pl.BlockSpec((BN, w), lambda i: (i, 0))
    out_specs = [blk(D), full((8, D))]
    out_shape = [jax.ShapeDtypeStruct((N, D), F32),
                 jax.ShapeDtypeStruct((8, D), F32)]
    if last:
        out_specs.append(blk(128))
        out_shape.append(jax.ShapeDtypeStruct((N, 128), F32))
    return pl.pallas_call(
        kern,
        grid=(nb,),
        in_specs=[
            blk(D), blk(D), blk(D), blk(c0.shape[1]), blk(c1.shape[1]),
            full(u8.shape), full(Wn1.shape), full(bn1.shape), full(W2x.shape),
            full(W2a.shape), full(W2u.shape), full(b20.shape),
            full(W21.shape), full(b21.shape), full(Ws1.shape),
            full(bs1.shape), full(Ws2p.shape), full(bs2p.shape),
        ],
        out_specs=out_specs,
        out_shape=out_shape,
    )(x, s0, s1, c0, c1, u8, Wn1, bn1, W2x, W2a, W2u, b20, W21, b21,
      Ws1, bs1, Ws2p, bs2p)


def _global(xs, es, u8, Wgx, Wge, Wgu, bg0, Wg1, bg1, Wg2, bg2, N, E):
    GD = Wg2.shape[1]

    def kern(xs_ref, es_ref, u_ref, wgx_ref, wge_ref, wgu_ref, bg0_ref,
             wg1_ref, bg1_ref, wg2_ref, bg2_ref, uo_ref):
        mx = xs_ref[...] * (1.0 / N)
        me = es_ref[...] * (1.0 / E)
        g = jnp.maximum(
            _dot(mx, wgx_ref[...]) + _dot(me, wge_ref[...]) +
            _dot(u_ref[...], wgu_ref[...]) + bg0_ref[0:1, :], 0.0)
        g = jnp.maximum(_dot(g, wg1_ref[...]) + bg1_ref[0:1, :], 0.0)
        uo_ref[...] = _dot(g, wg2_ref[...]) + bg2_ref[0:1, :]

    full = lambda a: pl.BlockSpec(a.shape, lambda: (0, 0))
    return pl.pallas_call(
        kern,
        grid=(),
        in_specs=[full(xs), full(es), full(u8), full(Wgx), full(Wge),
                  full(Wgu), full(bg0), full(Wg1), full(bg1), full(Wg2),
                  full(bg2)],
        out_specs=pl.BlockSpec((8, GD), lambda: (0, 0)),
        out_shape=jax.ShapeDtypeStruct((8, GD), F32),
    )(xs, es, u8, Wgx, Wge, Wgu, bg0, Wg1, bg1, Wg2, bg2)


def _sc_gather(r1t, rpt, ctab, row, col):
    N, H = r1t.shape
    E = row.shape[0]
    info = plsc.get_sparse_core_info()
    NC, NS = info.num_cores, info.num_subcores
    NW = NC * NS
    EPW = E // NW
    NCH = EPW // _CH
    mesh = plsc.VectorSubcoreMesh(core_axis_name="c", subcore_axis_name="s")

    assert NCH % 2 == 1 and NCH >= 3

    @functools.partial(
        pl.kernel, mesh=mesh,
        out_type=[jax.ShapeDtypeStruct((E, H), F32)] * 3,
        scratch_types=[
            pltpu.VMEM((_CH,), jnp.int32), pltpu.VMEM((_CH,), jnp.int32),
            pltpu.VMEM((_CH,), jnp.int32), pltpu.VMEM((_CH,), jnp.int32),
            pltpu.VMEM((_CH, H), F32), pltpu.VMEM((_CH, H), F32),
            pltpu.VMEM((_CH, H), F32), pltpu.VMEM((_CH, H), F32),
            pltpu.VMEM((_CH, H), F32), pltpu.VMEM((_CH, H), F32),
            pltpu.SemaphoreType.DMA, pltpu.SemaphoreType.DMA,
        ])
    def k(r1_hbm, rp_hbm, ctab_hbm, row_hbm, col_hbm,
          g1_hbm, gp_hbm, gc_hbm,
          idxr0, idxc0, idxr1, idxc1,
          buf10, bufp0, bufc0, buf11, bufp1, bufc1,
          semg0, semg1):
        wid = lax.axis_index("s") * NC + lax.axis_index("c")
        base = wid * EPW
        idxr = (idxr0, idxr1)
        idxc = (idxc0, idxc1)
        buf1 = (buf10, buf11)
        bufp = (bufp0, bufp1)
        bufc = (bufc0, bufc1)
        semg = (semg0, semg1)

        def start(g, b):
            off = base + g * _CH
            pltpu.sync_copy(row_hbm.at[pl.ds(off, _CH)], idxr[b])
            pltpu.sync_copy(col_hbm.at[pl.ds(off, _CH)], idxc[b])
            pltpu.async_copy(r1_hbm.at[idxr[b]], buf1[b], semg[b])
            pltpu.async_copy(rp_hbm.at[idxr[b]], bufp[b], semg[b])
            pltpu.async_copy(ctab_hbm.at[idxc[b]], bufc[b], semg[b])

        def drain(b):
            pltpu.make_async_copy(r1_hbm.at[pl.ds(0, _CH)], buf1[b],
                                  semg[b]).wait()
            pltpu.make_async_copy(rp_hbm.at[pl.ds(0, _CH)], bufp[b],
                                  semg[b]).wait()
            pltpu.make_async_copy(ctab_hbm.at[pl.ds(0, _CH)], bufc[b],
                                  semg[b]).wait()

        def writeback(g, b):
            off = base + g * _CH
            pltpu.sync_copy(buf1[b], g1_hbm.at[pl.ds(off, _CH)])
            pltpu.sync_copy(bufp[b], gp_hbm.at[pl.ds(off, _CH)])
            pltpu.sync_copy(bufc[b], gc_hbm.at[pl.ds(off, _CH)])

        start(0, 0)

        def body(j, carry):
            g0 = 2 * j
            start(g0 + 1, 1)
            drain(0)
            writeback(g0, 0)

            @pl.when(g0 + 2 < NCH)
            def _():
                start(g0 + 2, 0)

            drain(1)
            writeback(g0 + 1, 1)
            return carry

        lax.fori_loop(0, NCH // 2, body, 0)
        drain(0)
        writeback(NCH - 1, 0)

    return k(r1t, rpt, ctab, row, col)


def _sc_scatter(m1, col, zeros_nd):
    E, H = m1.shape
    N = zeros_nd.shape[0]
    info = plsc.get_sparse_core_info()
    NC, NS = info.num_cores, info.num_subcores
    NW = NC * NS
    EPW = E // NW
    NCH = EPW // _CH
    stripe = N // NS
    mesh = plsc.VectorSubcoreMesh(core_axis_name="c", subcore_axis_name="s")

    assert NCH % 2 == 1 and NCH >= 3

    @functools.partial(
        pl.kernel, mesh=mesh,
        out_type=jax.ShapeDtypeStruct((NC * N, H), F32),
        scratch_types=[
            pltpu.VMEM((_CH,), jnp.int32), pltpu.VMEM((_CH,), jnp.int32),
            pltpu.VMEM((_CH, H), F32), pltpu.VMEM((_CH, H), F32),
            pltpu.VMEM_SHARED((N, H), F32),
            pltpu.SemaphoreType.DMA, pltpu.SemaphoreType.DMA,
        ])
    def k(m1_hbm, col_hbm, z_hbm, out_hbm, idx0, idx1, buf0, buf1, s_sh,
          seml0, seml1):
        cid = lax.axis_index("c")
        sid = lax.axis_index("s")
        wid = sid * NC + cid
        base = wid * EPW
        idx = (idx0, idx1)
        buf = (buf0, buf1)
        seml = (seml0, seml1)
        pltpu.sync_copy(z_hbm.at[pl.ds(sid * stripe, stripe)],
                        s_sh.at[pl.ds(sid * stripe, stripe)])
        plsc.subcore_barrier()

        def start(g, b):
            off = base + g * _CH
            pltpu.async_copy(col_hbm.at[pl.ds(off, _CH)], idx[b], seml[b])
            pltpu.async_copy(m1_hbm.at[pl.ds(off, _CH)], buf[b], seml[b])

        def drain(b):
            pltpu.make_async_copy(col_hbm.at[pl.ds(0, _CH)], idx[b],
                                  seml[b]).wait()
            pltpu.make_async_copy(m1_hbm.at[pl.ds(0, _CH)], buf[b],
                                  seml[b]).wait()

        def add(b):
            pltpu.sync_copy(buf[b], s_sh.at[idx[b]], add=True)

        start(0, 0)

        def body(j, carry):
            start(2 * j + 1, 1)
            drain(0)
            add(0)

            @pl.when(2 * j + 2 < NCH)
            def _():
                start(2 * j + 2, 0)

            drain(1)
            add(1)
            return carry

        lax.fori_loop(0, NCH // 2, body, 0)
        drain(0)
        add(0)
        plsc.subcore_barrier()
        pltpu.sync_copy(s_sh.at[pl.ds(sid * stripe, stripe)],
                        out_hbm.at[pl.ds(cid * N + sid * stripe, stripe)])

    return k(m1, col, zeros_nd)


def _sc_counts(col, zeros_nw, ones_w):
    E = col.shape[0]
    N, W = zeros_nw.shape
    info = plsc.get_sparse_core_info()
    NC, NS = info.num_cores, info.num_subcores
    NW = NC * NS
    EPW = E // NW
    NCH = EPW // _CH
    stripe = N // NS
    mesh = plsc.VectorSubcoreMesh(core_axis_name="c", subcore_axis_name="s")

    @functools.partial(
        pl.kernel, mesh=mesh,
        out_type=jax.ShapeDtypeStruct((NC * N, W), F32),
        scratch_types=[
            pltpu.VMEM((_CH,), jnp.int32),
            pltpu.VMEM((_CH, W), F32),
            pltpu.VMEM_SHARED((N, W), F32),
        ])
    def k(col_hbm, z_hbm, ones_hbm, out_hbm, idx, buf, c_sh):
        cid = lax.axis_index("c")
        sid = lax.axis_index("s")
        wid = sid * NC + cid
        pltpu.sync_copy(z_hbm.at[pl.ds(sid * stripe, stripe)],
                        c_sh.at[pl.ds(sid * stripe, stripe)])
        pltpu.sync_copy(ones_hbm, buf)
        plsc.subcore_barrier()

        def body(i, carry):
            off = wid * EPW + i * _CH
            pltpu.sync_copy(col_hbm.at[pl.ds(off, _CH)], idx)
            pltpu.sync_copy(buf, c_sh.at[idx], add=True)
            return carry

        lax.fori_loop(0, NCH, body, 0)
        plsc.subcore_barrier()
        pltpu.sync_copy(c_sh.at[pl.ds(sid * stripe, stripe)],
                        out_hbm.at[pl.ds(cid * N + sid * stripe, stripe)])

    return k(col, zeros_nw, ones_w)


def kernel(x, edge_attr, u, params, edge_index, batch):
    x = x.astype(F32)
    edge_attr = edge_attr.astype(F32)
    N, D = x.shape
    E, ED = edge_attr.shape
    GD = u.shape[1]
    row = edge_index[0]
    col = edge_index[1]
    layers = params['layers']

    u8 = jnp.broadcast_to(u.astype(F32), (8, GD))
    Np = ((N + 127) // 128) * 128
    zeros_nd = jnp.zeros((Np, D), F32)
    ones_w = jnp.ones((_CH, 128), F32)

    cnt2 = _sc_counts(col, zeros_nd, ones_w)
    c0, c1 = cnt2[:N, :16], cnt2[Np:Np + N, :16]

    (Ws1, bs1), (Ws2, bs2) = params['shift_predictor']
    Ws2p = jnp.pad(Ws2, ((0, 0), (0, 127)))
    bs2p = jnp.zeros((8, 128), F32).at[:, 0].set(bs2[0])

    e = edge_attr
    shifts_full = None
    for l, lay in enumerate(layers):
        (W0, b0), (W1, b1), (W2, b2) = lay['edge_mlp']
        Wa, Wb = W0[:D], W0[D:2 * D]
        Wc = W0[2 * D:2 * D + ED]
        Wd = W0[2 * D + ED:]
        (Wn0, bn0), (Wn1, bn1) = lay['node_mlp_1']
        Wna, Wnb = Wn0[:D], Wn0[D:]
        (W20, b20), (W21, b21) = lay['node_mlp_2']
        W2x, W2a, W2u = W20[:D], W20[D:2 * D], W20[2 * D:]
        (Wg0, bg0), (Wg1, bg1), (Wg2, bg2) = lay['global_mlp']
        Wgx, Wge, Wgu = Wg0[:D], Wg0[D:D + ED], Wg0[D + ED:]
        last = (l == len(layers) - 1)

        r1t, rpt, ctab = _node_pre(x, u8, Wa, Wb, Wna, Wd, _r8(b0), _r8(bn0))
        g1, gp, gc = _sc_gather(r1t, rpt, ctab, row, col)
        m1, e, esum = _edge(g1, gp, gc, e, Wc, W1, _r8(b1), W2, _r8(b2), Wnb)
        s2 = _sc_scatter(m1, col, zeros_nd)
        res = _node_post(x, s2[:N], s2[Np:Np + N], c0, c1, u8, Wn1, _r8(bn1),
                         W2x, W2a, W2u, _r8(b20), W21, _r8(b21),
                         Ws1, _r8(bs1), Ws2p, bs2p, last)
        if last:
            x, xsum, shifts_full = res
        else:
            x, xsum = res
        u8 = _global(xsum, esum, u8, Wgx, Wge, Wgu, _r8(bg0), Wg1, _r8(bg1),
                     Wg2, _r8(bg2), N, E)

    shifts = shifts_full[:, 0:1]
    return (shifts, (x, e, u8[0:1, :]))


# speedup vs baseline: 1.1319x; 1.1319x over previous
import functools

import jax
import jax.numpy as jnp
from jax import lax
from jax.experimental import pallas as pl
from jax.experimental.pallas import tpu as pltpu
from jax.experimental.pallas import tpu_sc as plsc

F32 = jnp.float32
_CH = 80


def _dot(a, b):
    return jnp.dot(a, b, preferred_element_type=F32,
                   precision=jax.lax.Precision.DEFAULT)


def _r8(v):
    v = v.reshape(1, -1)
    return jnp.broadcast_to(v, (8, v.shape[1]))


def _node_pre(x, u8, Wa, Wb, Wna, Wd, b0, bn0):
    N, D = x.shape
    BN = 400
    nb = N // BN

    def kern(x_ref, u_ref, wa_ref, wb_ref, wna_ref, wd_ref, b0_ref, bn0_ref,
             r1_ref, rp_ref, c_ref):
        xb = x_ref[...]
        ce = _dot(u_ref[...], wd_ref[...]) + b0_ref[...]
        r1_ref[...] = _dot(xb, wa_ref[...]) + ce[0:1, :]
        rp_ref[...] = _dot(xb, wna_ref[...]) + bn0_ref[0:1, :]
        c_ref[...] = _dot(xb, wb_ref[...])

    full = lambda s: pl.BlockSpec(s, lambda i: (0, 0))
    blk = pl.BlockSpec((BN, D), lambda i: (i, 0))
    return pl.pallas_call(
        kern,
        grid=(nb,),
        in_specs=[
            blk,
            full(u8.shape), full(Wa.shape), full(Wb.shape), full(Wna.shape),
            full(Wd.shape), full(b0.shape), full(bn0.shape),
        ],
        out_specs=[blk, blk, blk],
        out_shape=[jax.ShapeDtypeStruct((N, D), F32)] * 3,
    )(x, u8, Wa, Wb, Wna, Wd, b0, bn0)


def _edge(g1, gp, gc, e, Wc, W1, b1, W2, b2, Wnb):
    E, H = g1.shape
    ED = e.shape[1]
    BE = 800
    nb = E // BE

    def kern(g1_ref, gp_ref, gc_ref, e_ref, wc_ref, w1_ref, b1_ref, w2_ref,
             b2_ref, wnb_ref, m1_ref, eo_ref, es_ref):
        h1 = jnp.maximum(
            g1_ref[...] + gc_ref[...] + _dot(e_ref[...], wc_ref[...]), 0.0)
        h2 = jnp.maximum(_dot(h1, w1_ref[...]) + b1_ref[0:1, :], 0.0)
        en = _dot(h2, w2_ref[...]) + b2_ref[0:1, :]
        m1_ref[...] = jnp.maximum(gp_ref[...] + _dot(en, wnb_ref[...]), 0.0)
        eo_ref[...] = en

        @pl.when(pl.program_id(0) == 0)
        def _():
            es_ref[...] = jnp.zeros_like(es_ref)

        es_ref[...] += jnp.broadcast_to(
            jnp.sum(en, axis=0, keepdims=True), (8, ED))

    full = lambda s: pl.BlockSpec(s, lambda i: (0, 0))
    blkh = pl.BlockSpec((BE, H), lambda i: (i, 0))
    return pl.pallas_call(
        kern,
        grid=(nb,),
        in_specs=[
            blkh, blkh, blkh,
            pl.BlockSpec((BE, ED), lambda i: (i, 0)),
            full(Wc.shape), full(W1.shape), full(b1.shape), full(W2.shape),
            full(b2.shape), full(Wnb.shape),
        ],
        out_specs=[
            blkh,
            pl.BlockSpec((BE, ED), lambda i: (i, 0)),
            full((8, ED)),
        ],
        out_shape=[
            jax.ShapeDtypeStruct((E, H), F32),
            jax.ShapeDtypeStruct((E, ED), F32),
            jax.ShapeDtypeStruct((8, ED), F32),
        ],
    )(g1, gp, gc, e, Wc, W1, b1, W2, b2, Wnb)


def _node_post(x, s0, s1, c0, c1, u8, Wn1, bn1, W2x, W2a, W2u, b20, W21, b21,
               Ws1, bs1, Ws2p, bs2p, last):
    N, D = x.shape
    BN = 400
    nb = N // BN

    def kern(x_ref, s0_ref, s1_ref, c0_ref, c1_ref, u_ref, wn1_ref, bn1_ref,
             w2x_ref, w2a_ref, w2u_ref, b20_ref, w21_ref, b21_ref, ws1_ref,
             bs1_ref, ws2_ref, bs2_ref, *outs):
        if last:
            xo_ref, xs_ref, sh_ref = outs
        else:
            xo_ref, xs_ref = outs
        xb = x_ref[...]
        cnt = c0_ref[...][:, 0:1] + c1_ref[...][:, 0:1]
        s = s0_ref[...] + s1_ref[...]
        denom = jnp.maximum(cnt, 1.0)
        agg = (_dot(s, wn1_ref[...]) + cnt * bn1_ref[0:1, :]) / denom
        cu = _dot(u_ref[...], w2u_ref[...]) + b20_ref[...]
        t = jnp.maximum(
            _dot(xb, w2x_ref[...]) + _dot(agg, w2a_ref[...]) + cu[0:1, :], 0.0)
        xn = _dot(t, w21_ref[...]) + b21_ref[0:1, :]
        xo_ref[...] = xn

        @pl.when(pl.program_id(0) == 0)
        def _():
            xs_ref[...] = jnp.zeros_like(xs_ref)

        xs_ref[...] += jnp.broadcast_to(
            jnp.sum(xn, axis=0, keepdims=True), (8, D))

        if last:
            t2 = jnp.maximum(_dot(xn, ws1_ref[...]) + bs1_ref[0:1, :], 0.0)
            sh_ref[...] = _dot(t2, ws2_ref[...]) + bs2_ref[0:1, :]

    full = lambda s: pl.BlockSpec(s, lambda i: (0, 0))
    blk = lambda w: pl.BlockSpec((BN, w), lambda i: (i, 0))
    out_specs = [blk(D), full((8, D))]
    out_shape = [jax.ShapeDtypeStruct((N, D), F32),
                 jax.ShapeDtypeStruct((8, D), F32)]
    if last:
        out_specs.append(blk(128))
        out_shape.append(jax.ShapeDtypeStruct((N, 128), F32))
    return pl.pallas_call(
        kern,
        grid=(nb,),
        in_specs=[
            blk(D), blk(D), blk(D), blk(c0.shape[1]), blk(c1.shape[1]),
            full(u8.shape), full(Wn1.shape), full(bn1.shape), full(W2x.shape),
            full(W2a.shape), full(W2u.shape), full(b20.shape),
            full(W21.shape), full(b21.shape), full(Ws1.shape),
            full(bs1.shape), full(Ws2p.shape), full(bs2p.shape),
        ],
        out_specs=out_specs,
        out_shape=out_shape,
    )(x, s0, s1, c0, c1, u8, Wn1, bn1, W2x, W2a, W2u, b20, W21, b21,
      Ws1, bs1, Ws2p, bs2p)


def _global(xs, es, u8, Wgx, Wge, Wgu, bg0, Wg1, bg1, Wg2, bg2, N, E):
    GD = Wg2.shape[1]

    def kern(xs_ref, es_ref, u_ref, wgx_ref, wge_ref, wgu_ref, bg0_ref,
             wg1_ref, bg1_ref, wg2_ref, bg2_ref, uo_ref):
        mx = xs_ref[...] * (1.0 / N)
        me = es_ref[...] * (1.0 / E)
        g = jnp.maximum(
            _dot(mx, wgx_ref[...]) + _dot(me, wge_ref[...]) +
            _dot(u_ref[...], wgu_ref[...]) + bg0_ref[0:1, :], 0.0)
        g = jnp.maximum(_dot(g, wg1_ref[...]) + bg1_ref[0:1, :], 0.0)
        uo_ref[...] = _dot(g, wg2_ref[...]) + bg2_ref[0:1, :]

    full = lambda a: pl.BlockSpec(a.shape, lambda: (0, 0))
    return pl.pallas_call(
        kern,
        grid=(),
        in_specs=[full(xs), full(es), full(u8), full(Wgx), full(Wge),
                  full(Wgu), full(bg0), full(Wg1), full(bg1), full(Wg2),
                  full(bg2)],
        out_specs=pl.BlockSpec((8, GD), lambda: (0, 0)),
        out_shape=jax.ShapeDtypeStruct((8, GD), F32),
    )(xs, es, u8, Wgx, Wge, Wgu, bg0, Wg1, bg1, Wg2, bg2)


def _sc_gather(r1t, rpt, ctab, row, col):
    N, H = r1t.shape
    E = row.shape[0]
    info = plsc.get_sparse_core_info()
    NC, NS = info.num_cores, info.num_subcores
    NW = NC * NS
    EPW = E // NW
    NCH = EPW // _CH
    mesh = plsc.VectorSubcoreMesh(core_axis_name="c", subcore_axis_name="s")

    assert NCH % 2 == 1 and NCH >= 3

    @functools.partial(
        pl.kernel, mesh=mesh,
        out_type=[jax.ShapeDtypeStruct((E, H), F32)] * 3,
        scratch_types=[
            pltpu.VMEM((_CH,), jnp.int32), pltpu.VMEM((_CH,), jnp.int32),
            pltpu.VMEM((_CH,), jnp.int32), pltpu.VMEM((_CH,), jnp.int32),
            pltpu.VMEM((_CH, H), F32), pltpu.VMEM((_CH, H), F32),
            pltpu.VMEM((_CH, H), F32), pltpu.VMEM((_CH, H), F32),
            pltpu.VMEM((_CH, H), F32), pltpu.VMEM((_CH, H), F32),
            pltpu.SemaphoreType.DMA, pltpu.SemaphoreType.DMA,
        ])
    def k(r1_hbm, rp_hbm, ctab_hbm, row_hbm, col_hbm,
          g1_hbm, gp_hbm, gc_hbm,
          idxr0, idxc0, idxr1, idxc1,
          buf10, bufp0, bufc0, buf11, bufp1, bufc1,
          semg0, semg1):
        wid = lax.axis_index("s") * NC + lax.axis_index("c")
        base = wid * EPW
        idxr = (idxr0, idxr1)
        idxc = (idxc0, idxc1)
        buf1 = (buf10, buf11)
        bufp = (bufp0, bufp1)
        bufc = (bufc0, bufc1)
        semg = (semg0, semg1)

        def start(g, b):
            off = base + g * _CH
            pltpu.sync_copy(row_hbm.at[pl.ds(off, _CH)], idxr[b])
            pltpu.sync_copy(col_hbm.at[pl.ds(off, _CH)], idxc[b])
            pltpu.async_copy(r1_hbm.at[idxr[b]], buf1[b], semg[b])
            pltpu.async_copy(rp_hbm.at[idxr[b]], bufp[b], semg[b])
            pltpu.async_copy(ctab_hbm.at[idxc[b]], bufc[b], semg[b])

        def drain(b):
            pltpu.make_async_copy(r1_hbm.at[pl.ds(0, _CH)], buf1[b],
                                  semg[b]).wait()
            pltpu.make_async_copy(rp_hbm.at[pl.ds(0, _CH)], bufp[b],
                                  semg[b]).wait()
            pltpu.make_async_copy(ctab_hbm.at[pl.ds(0, _CH)], bufc[b],
                                  semg[b]).wait()

        def writeback(g, b):
            off = base + g * _CH
            pltpu.sync_copy(buf1[b], g1_hbm.at[pl.ds(off, _CH)])
            pltpu.sync_copy(bufp[b], gp_hbm.at[pl.ds(off, _CH)])
            pltpu.sync_copy(bufc[b], gc_hbm.at[pl.ds(off, _CH)])

        start(0, 0)

        def body(j, carry):
            g0 = 2 * j
            start(g0 + 1, 1)
            drain(0)
            writeback(g0, 0)

            @pl.when(g0 + 2 < NCH)
            def _():
                start(g0 + 2, 0)

            drain(1)
            writeback(g0 + 1, 1)
            return carry

        lax.fori_loop(0, NCH // 2, body, 0)
        drain(0)
        writeback(NCH - 1, 0)

    return k(r1t, rpt, ctab, row, col)


def _sc_scatter(m1, col, zeros_nd):
    E, H = m1.shape
    N = zeros_nd.shape[0]
    info = plsc.get_sparse_core_info()
    NC, NS = info.num_cores, info.num_subcores
    NW = NC * NS
    EPW = E // NW
    NCH = EPW // _CH
    stripe = N // NS
    mesh = plsc.VectorSubcoreMesh(core_axis_name="c", subcore_axis_name="s")

    assert NCH % 2 == 1 and NCH >= 3

    @functools.partial(
        pl.kernel, mesh=mesh,
        out_type=jax.ShapeDtypeStruct((NC * N, H), F32),
        scratch_types=[
            pltpu.VMEM((_CH,), jnp.int32), pltpu.VMEM((_CH,), jnp.int32),
            pltpu.VMEM((_CH, H), F32), pltpu.VMEM((_CH, H), F32),
            pltpu.VMEM_SHARED((N, H), F32),
            pltpu.SemaphoreType.DMA, pltpu.SemaphoreType.DMA,
        ])
    def k(m1_hbm, col_hbm, z_hbm, out_hbm, idx0, idx1, buf0, buf1, s_sh,
          seml0, seml1):
        cid = lax.axis_index("c")
        sid = lax.axis_index("s")
        wid = sid * NC + cid
        base = wid * EPW
        idx = (idx0, idx1)
        buf = (buf0, buf1)
        seml = (seml0, seml1)
        pltpu.sync_copy(z_hbm.at[pl.ds(sid * stripe, stripe)],
                        s_sh.at[pl.ds(sid * stripe, stripe)])
        plsc.subcore_barrier()

        def start(g, b):
            off = base + g * _CH
            pltpu.async_copy(col_hbm.at[pl.ds(off, _CH)], idx[b], seml[b])
            pltpu.async_copy(m1_hbm.at[pl.ds(off, _CH)], buf[b], seml[b])

        def drain(b):
            pltpu.make_async_copy(col_hbm.at[pl.ds(0, _CH)], idx[b],
                                  seml[b]).wait()
            pltpu.make_async_copy(m1_hbm.at[pl.ds(0, _CH)], buf[b],
                                  seml[b]).wait()

        def add(b):
            pltpu.sync_copy(buf[b], s_sh.at[idx[b]], add=True)

        start(0, 0)

        def body(j, carry):
            start(2 * j + 1, 1)
            drain(0)
            add(0)

            @pl.when(2 * j + 2 < NCH)
            def _():
                start(2 * j + 2, 0)

            drain(1)
            add(1)
            return carry

        lax.fori_loop(0, NCH // 2, body, 0)
        drain(0)
        add(0)
        plsc.subcore_barrier()
        pltpu.sync_copy(s_sh.at[pl.ds(sid * stripe, stripe)],
                        out_hbm.at[pl.ds(cid * N + sid * stripe, stripe)])

    return k(m1, col, zeros_nd)


def _sc_counts(col, zeros_nw, ones_w):
    E = col.shape[0]
    N, W = zeros_nw.shape
    info = plsc.get_sparse_core_info()
    NC, NS = info.num_cores, info.num_subcores
    NW = NC * NS
    EPW = E // NW
    NCH = EPW // _CH
    stripe = N // NS
    mesh = plsc.VectorSubcoreMesh(core_axis_name="c", subcore_axis_name="s")

    @functools.partial(
        pl.kernel, mesh=mesh,
        out_type=jax.ShapeDtypeStruct((NC * N, W), F32),
        scratch_types=[
            pltpu.VMEM((_CH,), jnp.int32),
            pltpu.VMEM((_CH, W), F32),
            pltpu.VMEM_SHARED((N, W), F32),
        ])
    def k(col_hbm, z_hbm, ones_hbm, out_hbm, idx, buf, c_sh):
        cid = lax.axis_index("c")
        sid = lax.axis_index("s")
        wid = sid * NC + cid
        pltpu.sync_copy(z_hbm.at[pl.ds(sid * stripe, stripe)],
                        c_sh.at[pl.ds(sid * stripe, stripe)])
        pltpu.sync_copy(ones_hbm, buf)
        plsc.subcore_barrier()

        def body(i, carry):
            off = wid * EPW + i * _CH
            pltpu.sync_copy(col_hbm.at[pl.ds(off, _CH)], idx)
            pltpu.sync_copy(buf, c_sh.at[idx], add=True)
            return carry

        lax.fori_loop(0, NCH, body, 0)
        plsc.subcore_barrier()
        pltpu.sync_copy(c_sh.at[pl.ds(sid * stripe, stripe)],
                        out_hbm.at[pl.ds(cid * N + sid * stripe, stripe)])

    return k(col, zeros_nw, ones_w)


def kernel(x, edge_attr, u, params, edge_index, batch):
    x = x.astype(F32)
    edge_attr = edge_attr.astype(F32)
    N, D = x.shape
    E, ED = edge_attr.shape
    GD = u.shape[1]
    row = edge_index[0]
    col = edge_index[1]
    layers = params['layers']

    u8 = jnp.broadcast_to(u.astype(F32), (8, GD))
    Np = ((N + 127) // 128) * 128
    zeros_nd = jnp.zeros((Np, D), F32)
    ones_w = jnp.ones((_CH, 128), F32)

    cnt2 = _sc_counts(col, zeros_nd, ones_w)
    c0, c1 = cnt2[:N, :16], cnt2[Np:Np + N, :16]

    (Ws1, bs1), (Ws2, bs2) = params['shift_predictor']
    Ws2p = jnp.pad(Ws2, ((0, 0), (0, 127)))
    bs2p = jnp.zeros((8, 128), F32).at[:, 0].set(bs2[0])

    e = edge_attr
    shifts_full = None
    for l, lay in enumerate(layers):
        (W0, b0), (W1, b1), (W2, b2) = lay['edge_mlp']
        Wa, Wb = W0[:D], W0[D:2 * D]
        Wc = W0[2 * D:2 * D + ED]
        Wd = W0[2 * D + ED:]
        (Wn0, bn0), (Wn1, bn1) = lay['node_mlp_1']
        Wna, Wnb = Wn0[:D], Wn0[D:]
        (W20, b20), (W21, b21) = lay['node_mlp_2']
        W2x, W2a, W2u = W20[:D], W20[D:2 * D], W20[2 * D:]
        (Wg0, bg0), (Wg1, bg1), (Wg2, bg2) = lay['global_mlp']
        Wgx, Wge, Wgu = Wg0[:D], Wg0[D:D + ED], Wg0[D + ED:]
        last = (l == len(layers) - 1)

        r1t, rpt, ctab = _node_pre(x, u8, Wa, Wb, Wna, Wd, _r8(b0), _r8(bn0))
        g1, gp, gc = _sc_gather(r1t, rpt, ctab, row, col)
        m1, e, esum = _edge(g1, gp, gc, e, Wc, W1, _r8(b1), W2, _r8(b2), Wnb)
        s2 = _sc_scatter(m1, col, zeros_nd)
        res = _node_post(x, s2[:N], s2[Np:Np + N], c0, c1, u8, Wn1, _r8(bn1),
                         W2x, W2a, W2u, _r8(b20), W21, _r8(b21),
                         Ws1, _r8(bs1), Ws2p, bs2p, last)
        if last:
            x, xsum, shifts_full = res
        else:
            x, xsum = res
        u8 = _global(xsum, esum, u8, Wgx, Wge, Wgu, _r8(bg0), Wg1, _r8(bg1),
                     Wg2, _r8(bg2), N, E)

    shifts = shifts_full[:, 0:1]
    return (shifts, (x, e, u8[0:1, :]))
